# Initial kernel scaffold; baseline (speedup 1.0000x reference)
import jax
import jax.numpy as jnp
from jax.experimental import pallas as pl


def kernel(nodes, neigh_idx, features, weight):
    raise NotImplementedError("write your pallas kernel here")


# speedup vs baseline: 1.8656x; 1.8656x over previous
import functools

import jax
import jax.numpy as jnp
from jax import lax
from jax.experimental import pallas as pl
from jax.experimental.pallas import tpu as pltpu
from jax.experimental.pallas import tpu_sc as plsc

FEAT = 128
EMBED = 128
NSAMP = 10

NC = 2
NSUB = 16
NW = NC * NSUB
B_PAD = 51200
PER_W = B_PAD // NW
CHUNK = 32
NCHUNK = PER_W // CHUNK
NIDX = CHUNK * NSAMP

_SC_MESH = plsc.VectorSubcoreMesh(core_axis_name="c", subcore_axis_name="s")


def _sc_body(feat, nodes, neigh, self_out, sum_out,
             self_idx, neigh_idx, self_rows, neigh_rows, sum_rows, gsem):
    wid = lax.axis_index("s") * NC + lax.axis_index("c")
    base = wid * PER_W

    pltpu.sync_copy(nodes.at[pl.ds(base, PER_W)], self_idx)
    pltpu.sync_copy(neigh.at[pl.ds(base * NSAMP, PER_W * NSAMP)], neigh_idx)

    def chunk_body(c, carry):
        coff = pl.multiple_of(c * CHUNK, 8)
        noff = pl.multiple_of(c * NIDX, 8)
        pltpu.async_copy(feat.at[self_idx.at[pl.ds(coff, CHUNK)]],
                         self_rows, gsem)
        pltpu.async_copy(feat.at[neigh_idx.at[pl.ds(noff, 128)]],
                         neigh_rows.at[pl.ds(0, 128)], gsem)
        pltpu.async_copy(feat.at[neigh_idx.at[pl.ds(noff + 128, 128)]],
                         neigh_rows.at[pl.ds(128, 128)], gsem)
        pltpu.async_copy(feat.at[neigh_idx.at[pl.ds(noff + 256, 64)]],
                         neigh_rows.at[pl.ds(256, 64)], gsem)
        pltpu.make_async_copy(feat.at[pl.ds(0, CHUNK)], self_rows, gsem).wait()
        pltpu.make_async_copy(feat.at[pl.ds(0, NIDX)], neigh_rows, gsem).wait()

        def elem_body(e, carry2):
            rb = e * NSAMP
            for v in range(FEAT // 16):
                sl = pl.ds(v * 16, 16)
                acc = neigh_rows[rb, sl]
                for j in range(1, NSAMP):
                    acc = acc + neigh_rows[rb + j, sl]
                sum_rows[e, sl] = acc
            return carry2

        lax.fori_loop(0, CHUNK, elem_body, 0)

        out_off = pl.multiple_of(base + coff, 8)
        pltpu.sync_copy(self_rows, self_out.at[pl.ds(out_off, CHUNK)])
        pltpu.sync_copy(sum_rows, sum_out.at[pl.ds(out_off, CHUNK)])
        return carry

    lax.fori_loop(0, NCHUNK, chunk_body, 0)


@functools.partial(
    pl.kernel,
    out_type=(jax.ShapeDtypeStruct((B_PAD, FEAT), jnp.float32),
              jax.ShapeDtypeStruct((B_PAD, FEAT), jnp.float32)),
    mesh=_SC_MESH,
    scratch_types=[
        pltpu.VMEM((PER_W,), jnp.int32),
        pltpu.VMEM((PER_W * NSAMP,), jnp.int32),
        pltpu.VMEM((CHUNK, FEAT), jnp.float32),
        pltpu.VMEM((NIDX, FEAT), jnp.float32),
        pltpu.VMEM((CHUNK, FEAT), jnp.float32),
        pltpu.SemaphoreType.DMA,
    ],
)
def _sc_gather(feat, nodes, neigh, self_out, sum_out,
               self_idx, neigh_idx, self_rows, neigh_rows, sum_rows, gsem):
    _sc_body(feat, nodes, neigh, self_out, sum_out,
             self_idx, neigh_idx, self_rows, neigh_rows, sum_rows, gsem)


def _tc_matmul_body(w1_ref, w2_ref, s_ref, n_ref, o_ref):
    a = lax.dot_general(w1_ref[...], s_ref[...], (((1,), (1,)), ((), ())),
                        preferred_element_type=jnp.float32)
    b = lax.dot_general(w2_ref[...], n_ref[...], (((1,), (1,)), ((), ())),
                        preferred_element_type=jnp.float32)
    o_ref[...] = jnp.maximum(a + b, 0.0)


def _tc_matmul(w1, w2, self_rows, sum_rows, b_out, tb):
    nb = pl.cdiv(b_out, tb)
    return pl.pallas_call(
        _tc_matmul_body,
        grid=(nb,),
        in_specs=[
            pl.BlockSpec((EMBED, FEAT), lambda i: (0, 0)),
            pl.BlockSpec((EMBED, FEAT), lambda i: (0, 0)),
            pl.BlockSpec((tb, FEAT), lambda i: (i, 0)),
            pl.BlockSpec((tb, FEAT), lambda i: (i, 0)),
        ],
        out_specs=pl.BlockSpec((EMBED, tb), lambda i: (0, i)),
        out_shape=jax.ShapeDtypeStruct((EMBED, b_out), jnp.float32),
    )(w1, w2, self_rows, sum_rows)


def kernel(nodes, neigh_idx, features, weight):
    b = nodes.shape[0]
    pad = B_PAD - b
    nodes_p = jnp.concatenate([nodes, jnp.zeros((pad,), jnp.int32)])
    neigh_p = jnp.concatenate(
        [neigh_idx.reshape(-1), jnp.zeros((pad * NSAMP,), jnp.int32)])
    self_rows, sum_rows = _sc_gather(features, nodes_p, neigh_p)
    w1 = weight[:, :FEAT]
    w2 = weight[:, FEAT:] * (1.0 / NSAMP)
    return _tc_matmul(w1, w2, self_rows, sum_rows, b, 1024)


# speedup vs baseline: 2.2020x; 1.1803x over previous
import functools

import jax
import jax.numpy as jnp
from jax import lax
from jax.experimental import pallas as pl
from jax.experimental.pallas import tpu as pltpu
from jax.experimental.pallas import tpu_sc as plsc

FEAT = 128
EMBED = 128
NSAMP = 10

NC = 2
NSUB = 16
NW = NC * NSUB
B_PAD = 51200
PER_W = B_PAD // NW
CHUNK = 32
NCHUNK = PER_W // CHUNK
NIDX = CHUNK * NSAMP

_SC_MESH = plsc.VectorSubcoreMesh(core_axis_name="c", subcore_axis_name="s")


NPAIR = NCHUNK // 2


def _sc_body(feat, nodes, neigh, self_out, sum_out,
             self_idx, neigh_idx, self_rows, neigh_rows, sum_rows,
             gsem, ssem):
    wid = lax.axis_index("s") * NC + lax.axis_index("c")
    base = wid * PER_W

    pltpu.sync_copy(nodes.at[pl.ds(base, PER_W)], self_idx)
    pltpu.sync_copy(neigh.at[pl.ds(base * NSAMP, PER_W * NSAMP)], neigh_idx)

    def issue_gather(c, s):
        coff = pl.multiple_of(c * CHUNK, 8)
        noff = pl.multiple_of(c * NIDX, 8)
        pltpu.async_copy(feat.at[self_idx.at[pl.ds(coff, CHUNK)]],
                         self_rows.at[s], gsem.at[s])
        pltpu.async_copy(feat.at[neigh_idx.at[pl.ds(noff, 128)]],
                         neigh_rows.at[s, pl.ds(0, 128)], gsem.at[s])
        pltpu.async_copy(feat.at[neigh_idx.at[pl.ds(noff + 128, 128)]],
                         neigh_rows.at[s, pl.ds(128, 128)], gsem.at[s])
        pltpu.async_copy(feat.at[neigh_idx.at[pl.ds(noff + 256, 64)]],
                         neigh_rows.at[s, pl.ds(256, 64)], gsem.at[s])

    def wait_gather(s):
        pltpu.make_async_copy(feat.at[pl.ds(0, NIDX)], neigh_rows.at[s],
                              gsem.at[s]).wait()
        pltpu.make_async_copy(feat.at[pl.ds(0, CHUNK)], self_rows.at[s],
                              gsem.at[s]).wait()

    def compute(s):
        def elem_body(e, carry2):
            rb = e * NSAMP
            for v in range(FEAT // 16):
                sl = pl.ds(v * 16, 16)
                r = [neigh_rows[s, rb + j, sl] for j in range(NSAMP)]
                t01, t23 = r[0] + r[1], r[2] + r[3]
                t45, t67 = r[4] + r[5], r[6] + r[7]
                t89 = r[8] + r[9]
                sum_rows[s, e, sl] = ((t01 + t23) + (t45 + t67)) + t89
            return carry2

        lax.fori_loop(0, CHUNK, elem_body, 0)

    def issue_scatter(c, s):
        off = pl.multiple_of(base + c * CHUNK, 8)
        pltpu.async_copy(self_rows.at[s], self_out.at[pl.ds(off, CHUNK)],
                         ssem.at[s])
        pltpu.async_copy(sum_rows.at[s], sum_out.at[pl.ds(off, CHUNK)],
                         ssem.at[s])

    def wait_scatter(s):
        pltpu.make_async_copy(self_rows.at[s], self_out.at[pl.ds(0, CHUNK)],
                              ssem.at[s]).wait()
        pltpu.make_async_copy(sum_rows.at[s], sum_out.at[pl.ds(0, CHUNK)],
                              ssem.at[s]).wait()

    issue_gather(0, 0)

    def pair_body(t, carry):
        c0 = t * 2

        @pl.when(t > 0)
        def _():
            wait_scatter(1)

        issue_gather(c0 + 1, 1)
        wait_gather(0)
        compute(0)
        issue_scatter(c0, 0)
        wait_gather(1)

        @pl.when(t < NPAIR - 1)
        def _():
            wait_scatter(0)
            issue_gather(c0 + 2, 0)

        compute(1)
        issue_scatter(c0 + 1, 1)
        return carry

    lax.fori_loop(0, NPAIR, pair_body, 0)
    wait_scatter(0)
    wait_scatter(1)


@functools.partial(
    pl.kernel,
    out_type=(jax.ShapeDtypeStruct((B_PAD, FEAT), jnp.float32),
              jax.ShapeDtypeStruct((B_PAD, FEAT), jnp.float32)),
    mesh=_SC_MESH,
    scratch_types=[
        pltpu.VMEM((PER_W,), jnp.int32),
        pltpu.VMEM((PER_W * NSAMP,), jnp.int32),
        pltpu.VMEM((2, CHUNK, FEAT), jnp.float32),
        pltpu.VMEM((2, NIDX, FEAT), jnp.float32),
        pltpu.VMEM((2, CHUNK, FEAT), jnp.float32),
        pltpu.SemaphoreType.DMA((2,)),
        pltpu.SemaphoreType.DMA((2,)),
    ],
)
def _sc_gather(feat, nodes, neigh, self_out, sum_out,
               self_idx, neigh_idx, self_rows, neigh_rows, sum_rows,
               gsem, ssem):
    _sc_body(feat, nodes, neigh, self_out, sum_out,
             self_idx, neigh_idx, self_rows, neigh_rows, sum_rows,
             gsem, ssem)


def _tc_matmul_body(w1_ref, w2_ref, s_ref, n_ref, o_ref):
    a = lax.dot_general(w1_ref[...], s_ref[...], (((1,), (1,)), ((), ())),
                        preferred_element_type=jnp.float32)
    b = lax.dot_general(w2_ref[...], n_ref[...], (((1,), (1,)), ((), ())),
                        preferred_element_type=jnp.float32)
    o_ref[...] = jnp.maximum(a + b, 0.0)


def _tc_matmul(w1, w2, self_rows, sum_rows, b_out, tb):
    nb = pl.cdiv(b_out, tb)
    return pl.pallas_call(
        _tc_matmul_body,
        grid=(nb,),
        in_specs=[
            pl.BlockSpec((EMBED, FEAT), lambda i: (0, 0)),
            pl.BlockSpec((EMBED, FEAT), lambda i: (0, 0)),
            pl.BlockSpec((tb, FEAT), lambda i: (i, 0)),
            pl.BlockSpec((tb, FEAT), lambda i: (i, 0)),
        ],
        out_specs=pl.BlockSpec((EMBED, tb), lambda i: (0, i)),
        out_shape=jax.ShapeDtypeStruct((EMBED, b_out), jnp.float32),
    )(w1, w2, self_rows, sum_rows)


def kernel(nodes, neigh_idx, features, weight):
    b = nodes.shape[0]
    pad = B_PAD - b
    nodes_p = jnp.concatenate([nodes, jnp.zeros((pad,), jnp.int32)])
    neigh_p = jnp.concatenate(
        [neigh_idx.reshape(-1), jnp.zeros((pad * NSAMP,), jnp.int32)])
    self_rows, sum_rows = _sc_gather(features, nodes_p, neigh_p)
    w1 = weight[:, :FEAT]
    w2 = weight[:, FEAT:] * (1.0 / NSAMP)
    return _tc_matmul(w1, w2, self_rows, sum_rows, b, 1024)


# speedup vs baseline: 2.4754x; 1.1241x over previous
import functools

import jax
import jax.numpy as jnp
from jax import lax
from jax.experimental import pallas as pl
from jax.experimental.pallas import tpu as pltpu
from jax.experimental.pallas import tpu_sc as plsc

FEAT = 128
EMBED = 128
NSAMP = 10

NC = 2
NSUB = 16
B_PAD = 51200
CHUNK = 32
CH_SID = 100
K0 = 76
KMAX = 76
NIDX = CHUNK * NSAMP

_SC_MESH = plsc.VectorSubcoreMesh(core_axis_name="c", subcore_axis_name="s")


def _sc_body(feat, nodes, neigh, self_out, sum_out,
             self_idx, neigh_idx, self_rows, neigh_rows, sum_rows,
             gsem, ssem):
    cid = lax.axis_index("c")
    sid = lax.axis_index("s")
    n_ch = jnp.where(cid == 0, K0, CH_SID - K0)
    start = jnp.where(cid == 0, 0, K0)
    base = pl.multiple_of((sid * CH_SID + start) * CHUNK, 8)

    pltpu.sync_copy(nodes.at[pl.ds(base, KMAX * CHUNK)], self_idx)
    pltpu.sync_copy(neigh.at[pl.ds(base * NSAMP, KMAX * NIDX)], neigh_idx)

    def issue_gather(c, s):
        coff = pl.multiple_of(c * CHUNK, 8)
        noff = pl.multiple_of(c * NIDX, 8)
        pltpu.async_copy(feat.at[self_idx.at[pl.ds(coff, CHUNK)]],
                         self_rows.at[s], gsem.at[s])
        pltpu.async_copy(feat.at[neigh_idx.at[pl.ds(noff, 128)]],
                         neigh_rows.at[s, pl.ds(0, 128)], gsem.at[s])
        pltpu.async_copy(feat.at[neigh_idx.at[pl.ds(noff + 128, 128)]],
                         neigh_rows.at[s, pl.ds(128, 128)], gsem.at[s])
        pltpu.async_copy(feat.at[neigh_idx.at[pl.ds(noff + 256, 64)]],
                         neigh_rows.at[s, pl.ds(256, 64)], gsem.at[s])

    def wait_gather(s):
        pltpu.make_async_copy(feat.at[pl.ds(0, NIDX)], neigh_rows.at[s],
                              gsem.at[s]).wait()
        pltpu.make_async_copy(feat.at[pl.ds(0, CHUNK)], self_rows.at[s],
                              gsem.at[s]).wait()

    def compute(s):
        def elem_body(e, carry2):
            rb = e * NSAMP
            for v in range(FEAT // 16):
                sl = pl.ds(v * 16, 16)
                r = [neigh_rows[s, rb + j, sl] for j in range(NSAMP)]
                t01, t23 = r[0] + r[1], r[2] + r[3]
                t45, t67 = r[4] + r[5], r[6] + r[7]
                t89 = r[8] + r[9]
                sum_rows[s, e, sl] = ((t01 + t23) + (t45 + t67)) + t89
            return carry2

        lax.fori_loop(0, CHUNK, elem_body, 0)

    def issue_scatter(c, s):
        off = pl.multiple_of(base + c * CHUNK, 8)
        pltpu.async_copy(self_rows.at[s], self_out.at[pl.ds(off, CHUNK)],
                         ssem.at[s])
        pltpu.async_copy(sum_rows.at[s], sum_out.at[pl.ds(off, CHUNK)],
                         ssem.at[s])

    def wait_scatter(s):
        pltpu.make_async_copy(self_rows.at[s], self_out.at[pl.ds(0, CHUNK)],
                              ssem.at[s]).wait()
        pltpu.make_async_copy(sum_rows.at[s], sum_out.at[pl.ds(0, CHUNK)],
                              ssem.at[s]).wait()

    n_pair = n_ch // 2
    issue_gather(0, 0)

    def pair_body(t, carry):
        c0 = t * 2

        @pl.when(t > 0)
        def _():
            wait_scatter(1)

        issue_gather(c0 + 1, 1)
        wait_gather(0)
        compute(0)
        issue_scatter(c0, 0)
        wait_gather(1)

        @pl.when(t < n_pair - 1)
        def _():
            wait_scatter(0)
            issue_gather(c0 + 2, 0)

        compute(1)
        issue_scatter(c0 + 1, 1)
        return carry

    lax.fori_loop(0, n_pair, pair_body, 0)
    wait_scatter(0)
    wait_scatter(1)


@functools.partial(
    pl.kernel,
    out_type=(jax.ShapeDtypeStruct((B_PAD, FEAT), jnp.float32),
              jax.ShapeDtypeStruct((B_PAD, FEAT), jnp.float32)),
    mesh=_SC_MESH,
    scratch_types=[
        pltpu.VMEM((KMAX * CHUNK,), jnp.int32),
        pltpu.VMEM((KMAX * NIDX,), jnp.int32),
        pltpu.VMEM((2, CHUNK, FEAT), jnp.float32),
        pltpu.VMEM((2, NIDX, FEAT), jnp.float32),
        pltpu.VMEM((2, CHUNK, FEAT), jnp.float32),
        pltpu.SemaphoreType.DMA((2,)),
        pltpu.SemaphoreType.DMA((2,)),
    ],
)
def _sc_gather(feat, nodes, neigh, self_out, sum_out,
               self_idx, neigh_idx, self_rows, neigh_rows, sum_rows,
               gsem, ssem):
    _sc_body(feat, nodes, neigh, self_out, sum_out,
             self_idx, neigh_idx, self_rows, neigh_rows, sum_rows,
             gsem, ssem)


def _tc_matmul_body(w1_ref, w2_ref, s_ref, n_ref, o_ref):
    a = lax.dot_general(w1_ref[...], s_ref[...], (((1,), (1,)), ((), ())),
                        preferred_element_type=jnp.float32)
    b = lax.dot_general(w2_ref[...], n_ref[...], (((1,), (1,)), ((), ())),
                        preferred_element_type=jnp.float32)
    o_ref[...] = jnp.maximum(a + b, 0.0)


def _tc_matmul(w1, w2, self_rows, sum_rows, b_out, tb):
    nb = pl.cdiv(b_out, tb)
    return pl.pallas_call(
        _tc_matmul_body,
        grid=(nb,),
        in_specs=[
            pl.BlockSpec((EMBED, FEAT), lambda i: (0, 0)),
            pl.BlockSpec((EMBED, FEAT), lambda i: (0, 0)),
            pl.BlockSpec((tb, FEAT), lambda i: (i, 0)),
            pl.BlockSpec((tb, FEAT), lambda i: (i, 0)),
        ],
        out_specs=pl.BlockSpec((EMBED, tb), lambda i: (0, i)),
        out_shape=jax.ShapeDtypeStruct((EMBED, b_out), jnp.float32),
    )(w1, w2, self_rows, sum_rows)


def kernel(nodes, neigh_idx, features, weight):
    b = nodes.shape[0]
    pad = B_PAD - b
    slack = KMAX * CHUNK
    nodes_p = jnp.concatenate([nodes, jnp.zeros((pad + slack,), jnp.int32)])
    neigh_p = jnp.concatenate(
        [neigh_idx.reshape(-1),
         jnp.zeros(((pad + slack) * NSAMP,), jnp.int32)])
    self_rows, sum_rows = _sc_gather(features, nodes_p, neigh_p)
    w1 = weight[:, :FEAT]
    w2 = weight[:, FEAT:] * (1.0 / NSAMP)
    return _tc_matmul(w1, w2, self_rows, sum_rows, b, 1024)
